# Initial kernel scaffold; baseline (speedup 1.0000x reference)
import jax
import jax.numpy as jnp
from jax.experimental import pallas as pl


def kernel(x, vq, conv0_w, conv0_b, res1_w1, res1_b1, res1_w2, res1_b2, res2_w1, res2_b1, res2_w2, res2_b2, final_w, final_b, ln_g, ln_b, cur_iter):
    raise NotImplementedError("write your pallas kernel here")


# speedup vs baseline: 1.0194x; 1.0194x over previous
import functools

import jax
import jax.numpy as jnp
from jax import lax
from jax.experimental import pallas as pl
from jax.experimental.pallas import tpu as pltpu
from jax.experimental.pallas import tpu_sc as plsc

_TEMP = 0.1
_SC_NC, _SC_NS = 2, 16
_NW = _SC_NC * _SC_NS


def _norm_body(vq_ref, out_ref):
    v = vq_ref[...]
    n = jnp.sqrt(jnp.sum(v * v, axis=1, keepdims=True))
    out_ref[...] = v / jnp.maximum(n, 1e-12)


def _normalize_code(vq):
    K, C = vq.shape
    return pl.pallas_call(
        _norm_body,
        out_shape=jax.ShapeDtypeStruct((K, C), jnp.float32),
    )(vq)


def _vq_body(flat_ref, codeT_ref, dist_ref, asgT_ref, idx_ref):
    K = codeT_ref.shape[1]
    fl = flat_ref[...]
    norm = jnp.sqrt(jnp.sum(fl * fl, axis=1, keepdims=True))
    fln = fl / jnp.maximum(norm, 1e-12)
    fs = jnp.sum(fln * fln, axis=1, keepdims=True)
    ct = codeT_ref[...]
    cs = jnp.sum(ct * ct, axis=0, keepdims=True)
    dot = jnp.dot(fln, ct, preferred_element_type=jnp.float32)
    dist = fs + cs - 2.0 * dot
    dist_ref[...] = dist
    m = jnp.min(dist, axis=1, keepdims=True)
    cols = lax.broadcasted_iota(jnp.int32, dist.shape, 1)
    idx_ref[0, 0] = jnp.min(jnp.where(dist <= m, cols, K), axis=1).astype(jnp.int32)
    s = dist * (-1.0 / _TEMP)
    e = jnp.exp(s - jnp.max(s, axis=1, keepdims=True))
    p = e / jnp.sum(e, axis=1, keepdims=True)
    asgT_ref[0] = p.T


def _vq_call(flat, codeT, batch):
    N, C = flat.shape
    K = codeT.shape[1]
    R = 128
    nblk = N // R
    per_b = (N // batch) // R
    return pl.pallas_call(
        _vq_body,
        grid=(nblk,),
        in_specs=[
            pl.BlockSpec((R, C), lambda i: (i, 0)),
            pl.BlockSpec((C, K), lambda i: (0, 0)),
        ],
        out_specs=[
            pl.BlockSpec((R, K), lambda i: (i, 0)),
            pl.BlockSpec((1, K, R), lambda i: (i // per_b, 0, i % per_b)),
            pl.BlockSpec((1, 1, R), lambda i: (i, 0, 0)),
        ],
        out_shape=[
            jax.ShapeDtypeStruct((N, K), jnp.float32),
            jax.ShapeDtypeStruct((batch, K, N // batch), jnp.float32),
            jax.ShapeDtypeStruct((nblk, 1, R), jnp.int32),
        ],
    )(flat, codeT)


def _sc_gather(code_n, idx):
    K, C = code_n.shape
    N = idx.shape[0]
    bpw = N // _NW
    mesh = plsc.VectorSubcoreMesh(core_axis_name="c", subcore_axis_name="s")

    @functools.partial(
        pl.kernel,
        mesh=mesh,
        out_type=jax.ShapeDtypeStruct((N, C), jnp.float32),
        scratch_types=[
            pltpu.VMEM((bpw,), jnp.int32),
            pltpu.VMEM((bpw, C), jnp.float32),
            pltpu.SemaphoreType.DMA,
        ],
    )
    def gk(table_hbm, idx_hbm, out_hbm, idx_v, rows_v, sem):
        wid = lax.axis_index("s") * _SC_NC + lax.axis_index("c")
        base = wid * bpw
        pltpu.sync_copy(idx_hbm.at[pl.ds(base, bpw)], idx_v)
        pltpu.async_copy(table_hbm.at[idx_v], rows_v, sem).wait()
        pltpu.sync_copy(rows_v, out_hbm.at[pl.ds(base, bpw)])

    return gk(code_n, idx)


def _conv3x3(x, w_ref, col):
    HW, Cin = x.shape
    acc = None
    for j in range(9):
        dy, dx = j // 3 - 1, j % 3 - 1
        s = 32 * dy + dx
        if s > 0:
            patch = jnp.concatenate([x[s:], jnp.zeros((s, Cin), jnp.float32)], axis=0)
        elif s < 0:
            patch = jnp.concatenate([jnp.zeros((-s, Cin), jnp.float32), x[: HW + s]], axis=0)
        else:
            patch = x
        if dx == 1:
            patch = jnp.where(col != 31, patch, 0.0)
        elif dx == -1:
            patch = jnp.where(col != 0, patch, 0.0)
        d = jnp.dot(patch, w_ref[j], preferred_element_type=jnp.float32)
        acc = d if acc is None else acc + d
    return acc


def _dec_body(q_ref, w0_ref, b0_ref, w11_ref, b11_ref, w12_ref, b12_ref,
              w21_ref, b21_ref, w22_ref, b22_ref, wf_ref, bf_ref,
              g_ref, beta_ref, out_ref):
    x = q_ref[0]
    col = lax.broadcasted_iota(jnp.int32, (x.shape[0], 1), 0) % 32
    h = _conv3x3(x, w0_ref, col) + b0_ref[...]
    for w1_ref, b1_ref, w2_ref, b2_ref in ((w11_ref, b11_ref, w12_ref, b12_ref),
                                           (w21_ref, b21_ref, w22_ref, b22_ref)):
        r = _conv3x3(jnp.maximum(h, 0.0), w1_ref, col) + b1_ref[...]
        r = _conv3x3(jnp.maximum(r, 0.0), w2_ref, col) + b2_ref[...]
        h = h + r
    y = jnp.dot(h, wf_ref[...], preferred_element_type=jnp.float32) + bf_ref[...]
    mu = jnp.mean(y, axis=1, keepdims=True)
    yc = y - mu
    var = jnp.mean(yc * yc, axis=1, keepdims=True)
    out_ref[0] = yc / jnp.sqrt(var + 1e-6) * g_ref[...] + beta_ref[...]


def _dec_call(q, w0, b0, w11, b11, w12, b12, w21, b21, w22, b22, wf, bf, g, beta):
    B, HW, C = q.shape
    O = wf.shape[1]
    wspec = lambda a: pl.BlockSpec(a.shape, lambda i: (0,) * a.ndim)
    return pl.pallas_call(
        _dec_body,
        grid=(B,),
        in_specs=[pl.BlockSpec((1, HW, C), lambda i: (i, 0, 0))]
        + [wspec(a) for a in (w0, b0, w11, b11, w12, b12, w21, b21, w22, b22, wf, bf, g, beta)],
        out_specs=pl.BlockSpec((1, HW, O), lambda i: (i, 0, 0)),
        out_shape=jax.ShapeDtypeStruct((B, HW, O), jnp.float32),
    )(q, w0, b0, w11, b11, w12, b12, w21, b21, w22, b22, wf, bf, g, beta)


def _wmat(w):
    return jnp.transpose(w, (2, 3, 1, 0)).reshape(9, w.shape[1], w.shape[0])


def kernel(x, vq, conv0_w, conv0_b, res1_w1, res1_b1, res1_w2, res1_b2,
           res2_w1, res2_b1, res2_w2, res2_b2, final_w, final_b, ln_g, ln_b,
           cur_iter):
    B, C, H, W = x.shape
    K = vq.shape[0]
    flat = jnp.transpose(x, (0, 2, 3, 1)).reshape(B * H * W, C)
    code_n = _normalize_code(vq)
    dist, asgT, idx3 = _vq_call(flat, code_n.T, B)
    idx = idx3.reshape(B * H * W)
    q = _sc_gather(code_n, idx)
    qx = jnp.transpose(q.reshape(B, H, W, C), (0, 3, 1, 2))
    assignment = asgT.reshape(B, K, H, W)
    recon_nhwc = _dec_call(
        q.reshape(B, H * W, C),
        _wmat(conv0_w), conv0_b[None],
        _wmat(res1_w1), res1_b1[None], _wmat(res1_w2), res1_b2[None],
        _wmat(res2_w1), res2_b1[None], _wmat(res2_w2), res2_b2[None],
        final_w.reshape(final_w.shape[0], C).T, final_b[None],
        ln_g[None], ln_b[None],
    )
    Co = final_w.shape[0]
    recon = jnp.transpose(recon_nhwc.reshape(B, H, W, Co), (0, 3, 1, 2))
    return (qx, assignment, dist, recon)


# speedup vs baseline: 1.0270x; 1.0075x over previous
import functools

import jax
import jax.numpy as jnp
from jax import lax
from jax.experimental import pallas as pl
from jax.experimental.pallas import tpu as pltpu
from jax.experimental.pallas import tpu_sc as plsc

_TEMP = 0.1
_SC_NC, _SC_NS = 2, 16
_NW = _SC_NC * _SC_NS
_AUG = 8


def _prep_body(vq_ref, cn_ref, aug_ref):
    v = vq_ref[...]
    K = v.shape[0]
    norm = jnp.sqrt(jnp.sum(v * v, axis=1, keepdims=True))
    cn = v / jnp.maximum(norm, 1e-12)
    cn_ref[...] = cn
    cnT = cn.T
    cs = jnp.sum(cnT * cnT, axis=0, keepdims=True)
    aug_ref[...] = jnp.concatenate(
        [cnT, jnp.ones((1, K), jnp.float32), cs,
         jnp.zeros((_AUG - 2, K), jnp.float32)], axis=0)


def _prep_code(vq):
    K, C = vq.shape
    return pl.pallas_call(
        _prep_body,
        out_shape=[
            jax.ShapeDtypeStruct((K, C), jnp.float32),
            jax.ShapeDtypeStruct((C + _AUG, K), jnp.float32),
        ],
    )(vq)


def _vq_body(x_ref, aug_ref, dist_ref, asgT_ref, idx_ref):
    K = aug_ref.shape[1]
    R = x_ref.shape[2]
    fl = x_ref[0].T
    norm = jnp.sqrt(jnp.sum(fl * fl, axis=1, keepdims=True))
    fln = fl / jnp.maximum(norm, 1e-12)
    fs = jnp.sum(fln * fln, axis=1, keepdims=True)
    a = jnp.concatenate(
        [fln * -2.0, fs, jnp.ones((R, 1), jnp.float32),
         jnp.zeros((R, _AUG - 2), jnp.float32)], axis=1)
    dist = jnp.dot(a, aug_ref[...], preferred_element_type=jnp.float32)
    dist_ref[...] = dist
    m = jnp.min(dist, axis=1, keepdims=True)
    cols = lax.broadcasted_iota(jnp.int32, dist.shape, 1)
    idx_ref[0, 0] = jnp.min(jnp.where(dist <= m, cols, K), axis=1).astype(jnp.int32)
    e = jnp.exp((m - dist) * (1.0 / _TEMP))
    p = e * (1.0 / jnp.sum(e, axis=1, keepdims=True))
    asgT_ref[0] = p.T


def _vq_call(xr, aug, batch):
    B, C, HW = xr.shape
    N = B * HW
    K = aug.shape[1]
    R = 128
    nblk = N // R
    per_b = HW // R
    return pl.pallas_call(
        _vq_body,
        grid=(nblk,),
        in_specs=[
            pl.BlockSpec((1, C, R), lambda i: (i // per_b, 0, i % per_b)),
            pl.BlockSpec((C + _AUG, K), lambda i: (0, 0)),
        ],
        out_specs=[
            pl.BlockSpec((R, K), lambda i: (i, 0)),
            pl.BlockSpec((1, K, R), lambda i: (i // per_b, 0, i % per_b)),
            pl.BlockSpec((1, 1, R), lambda i: (i, 0, 0)),
        ],
        out_shape=[
            jax.ShapeDtypeStruct((N, K), jnp.float32),
            jax.ShapeDtypeStruct((batch, K, HW), jnp.float32),
            jax.ShapeDtypeStruct((nblk, 1, R), jnp.int32),
        ],
    )(xr, aug)


def _sc_gather(code_n, idx):
    K, C = code_n.shape
    N = idx.shape[0]
    bpw = N // _NW
    mesh = plsc.VectorSubcoreMesh(core_axis_name="c", subcore_axis_name="s")

    @functools.partial(
        pl.kernel,
        mesh=mesh,
        out_type=jax.ShapeDtypeStruct((N, C), jnp.float32),
        scratch_types=[
            pltpu.VMEM((bpw,), jnp.int32),
            pltpu.VMEM((bpw, C), jnp.float32),
            pltpu.SemaphoreType.DMA,
        ],
    )
    def gk(table_hbm, idx_hbm, out_hbm, idx_v, rows_v, sem):
        wid = lax.axis_index("s") * _SC_NC + lax.axis_index("c")
        base = wid * bpw
        pltpu.sync_copy(idx_hbm.at[pl.ds(base, bpw)], idx_v)
        pltpu.async_copy(table_hbm.at[idx_v], rows_v, sem).wait()
        pltpu.sync_copy(rows_v, out_hbm.at[pl.ds(base, bpw)])

    return gk(code_n, idx)


def _conv3x3(x, w_ref, col):
    HW, Cin = x.shape
    acc = None
    for j in range(9):
        dy, dx = j // 3 - 1, j % 3 - 1
        s = 32 * dy + dx
        if s > 0:
            patch = jnp.concatenate([x[s:], jnp.zeros((s, Cin), jnp.float32)], axis=0)
        elif s < 0:
            patch = jnp.concatenate([jnp.zeros((-s, Cin), jnp.float32), x[: HW + s]], axis=0)
        else:
            patch = x
        if dx == 1:
            patch = jnp.where(col != 31, patch, 0.0)
        elif dx == -1:
            patch = jnp.where(col != 0, patch, 0.0)
        d = jnp.dot(patch, w_ref[j], preferred_element_type=jnp.float32)
        acc = d if acc is None else acc + d
    return acc


def _dec_body(q_ref, w0_ref, b0_ref, w11_ref, b11_ref, w12_ref, b12_ref,
              w21_ref, b21_ref, w22_ref, b22_ref, wf_ref, bf_ref,
              g_ref, beta_ref, out_ref, qx_ref):
    x = q_ref[0]
    qx_ref[0] = x.T
    col = lax.broadcasted_iota(jnp.int32, (x.shape[0], 1), 0) % 32
    h = _conv3x3(x, w0_ref, col) + b0_ref[...]
    for w1_ref, b1_ref, w2_ref, b2_ref in ((w11_ref, b11_ref, w12_ref, b12_ref),
                                           (w21_ref, b21_ref, w22_ref, b22_ref)):
        r = _conv3x3(jnp.maximum(h, 0.0), w1_ref, col) + b1_ref[...]
        r = _conv3x3(jnp.maximum(r, 0.0), w2_ref, col) + b2_ref[...]
        h = h + r
    y = jnp.dot(h, wf_ref[...], preferred_element_type=jnp.float32) + bf_ref[...]
    mu = jnp.mean(y, axis=1, keepdims=True)
    yc = y - mu
    var = jnp.mean(yc * yc, axis=1, keepdims=True)
    out_ref[0] = (yc / jnp.sqrt(var + 1e-6) * g_ref[...] + beta_ref[...]).T


def _dec_call(q, w0, b0, w11, b11, w12, b12, w21, b21, w22, b22, wf, bf, g, beta):
    B, HW, C = q.shape
    O = wf.shape[1]
    wspec = lambda a: pl.BlockSpec(a.shape, lambda i: (0,) * a.ndim)
    return pl.pallas_call(
        _dec_body,
        grid=(B,),
        in_specs=[pl.BlockSpec((1, HW, C), lambda i: (i, 0, 0))]
        + [wspec(a) for a in (w0, b0, w11, b11, w12, b12, w21, b21, w22, b22, wf, bf, g, beta)],
        out_specs=[
            pl.BlockSpec((1, O, HW), lambda i: (i, 0, 0)),
            pl.BlockSpec((1, C, HW), lambda i: (i, 0, 0)),
        ],
        out_shape=[
            jax.ShapeDtypeStruct((B, O, HW), jnp.float32),
            jax.ShapeDtypeStruct((B, C, HW), jnp.float32),
        ],
    )(q, w0, b0, w11, b11, w12, b12, w21, b21, w22, b22, wf, bf, g, beta)


def _wmat(w):
    return jnp.transpose(w, (2, 3, 1, 0)).reshape(9, w.shape[1], w.shape[0])


def kernel(x, vq, conv0_w, conv0_b, res1_w1, res1_b1, res1_w2, res1_b2,
           res2_w1, res2_b1, res2_w2, res2_b2, final_w, final_b, ln_g, ln_b,
           cur_iter):
    B, C, H, W = x.shape
    K = vq.shape[0]
    code_n, aug = _prep_code(vq)
    dist, asgT, idx3 = _vq_call(x.reshape(B, C, H * W), aug, B)
    idx = idx3.reshape(B * H * W)
    q = _sc_gather(code_n, idx)
    assignment = asgT.reshape(B, K, H, W)
    recon_t, qx_t = _dec_call(
        q.reshape(B, H * W, C),
        _wmat(conv0_w), conv0_b[None],
        _wmat(res1_w1), res1_b1[None], _wmat(res1_w2), res1_b2[None],
        _wmat(res2_w1), res2_b1[None], _wmat(res2_w2), res2_b2[None],
        final_w.reshape(final_w.shape[0], C).T, final_b[None],
        ln_g[None], ln_b[None],
    )
    qx = qx_t.reshape(B, C, H, W)
    recon = recon_t.reshape(B, final_w.shape[0], H, W)
    return (qx, assignment, dist, recon)
